# baseline (speedup 1.0000x reference)
import functools

import jax
import jax.numpy as jnp
from jax import lax
from jax.experimental import pallas as pl
from jax.experimental.pallas import tpu as pltpu
from jax.experimental.pallas import tpu_sc as plsc


@functools.lru_cache(maxsize=None)
def _build_gather(V, D, B):
    info = plsc.get_sparse_core_info()
    NC, NS, L = info.num_cores, info.num_subcores, info.num_lanes
    NW = NC * NS
    assert B % NW == 0 and D % L == 0
    per_w = B // NW
    GR = L
    n_g = per_w // GR
    mesh = plsc.VectorSubcoreMesh(core_axis_name="c", subcore_axis_name="s")

    @functools.partial(
        pl.kernel,
        mesh=mesh,
        out_type=jax.ShapeDtypeStruct((B, D), jnp.float32),
        scratch_types=[
            pltpu.VMEM((per_w,), jnp.int32),
            pltpu.SemaphoreType.DMA,
        ],
    )
    def gather(x_hbm, ow_hbm, nw_hbm, out_hbm, idx_v, gsem):
        wid = lax.axis_index("s") * NC + lax.axis_index("c")
        base = wid * per_w
        pltpu.sync_copy(x_hbm.at[pl.ds(base, per_w)], idx_v)

        def issue(g, _):
            ivec = jnp.minimum(idx_v[pl.ds(g * GR, GR)], V - 1)
            for r in range(GR):
                pltpu.async_copy(
                    ow_hbm.at[ivec[r]], out_hbm.at[base + g * GR + r], gsem)
            return 0

        lax.fori_loop(0, n_g, issue, 0)

        def drain(g, _):
            pltpu.make_async_copy(
                ow_hbm.at[pl.ds(0, GR)], out_hbm.at[pl.ds(0, GR)], gsem
            ).wait()
            return 0

        lax.fori_loop(0, n_g, drain, 0)

        def fix_body(h, _):
            ivec = idx_v[pl.ds(h * L, L)]
            for r in range(L):
                iv = ivec[r]

                @pl.when(iv >= V)
                def _():
                    pltpu.sync_copy(
                        nw_hbm.at[iv - (V - 1)], out_hbm.at[base + h * L + r])
            return 0

        lax.fori_loop(0, per_w // L, fix_body, 0)

    return gather


def _cvt_body(x_ref, o_ref):
    o_ref[...] = x_ref[...].astype(jnp.int32)


@functools.lru_cache(maxsize=None)
def _build_convert(B, D, blk):
    return pl.pallas_call(
        _cvt_body,
        grid=(pl.cdiv(B, blk),),
        in_specs=[pl.BlockSpec((blk, D), lambda i: (i, 0))],
        out_specs=pl.BlockSpec((blk, D), lambda i: (i, 0)),
        out_shape=jax.ShapeDtypeStruct((B, D), jnp.int32),
    )


def kernel(x, original_weight, new_weight):
    V, D = original_weight.shape
    Bt, S = x.shape
    B = Bt * S
    rows = _build_gather(V, D, B)(x.reshape(B), original_weight, new_weight)
    out = _build_convert(B, D, 512)(rows)
    return out.reshape(Bt, S, D)


# speedup vs baseline: 10.3706x; 10.3706x over previous
import functools

import jax
import jax.numpy as jnp
from jax import lax
from jax.experimental import pallas as pl
from jax.experimental.pallas import tpu as pltpu
from jax.experimental.pallas import tpu_sc as plsc


@functools.lru_cache(maxsize=None)
def _build_gather(V, D, B):
    info = plsc.get_sparse_core_info()
    NC, NS, L = info.num_cores, info.num_subcores, info.num_lanes
    NW = NC * NS
    assert B % NW == 0 and D % L == 0
    per_w = B // NW
    GR = L
    NB = 4
    n_g = per_w // GR
    assert n_g >= NB
    mesh = plsc.VectorSubcoreMesh(core_axis_name="c", subcore_axis_name="s")

    @functools.partial(
        pl.kernel,
        mesh=mesh,
        out_type=jax.ShapeDtypeStruct((B, D), jnp.float32),
        scratch_types=[
            pltpu.VMEM((per_w,), jnp.int32),
        ]
        + [pltpu.VMEM((GR, D), jnp.float32) for _ in range(NB)]
        + [pltpu.SemaphoreType.DMA for _ in range(2 * NB)],
    )
    def gather(x_hbm, ow_hbm, nw_hbm, out_hbm, idx_v, *scr):
        bufs = scr[:NB]
        gsems = scr[NB:2 * NB]
        ssems = scr[2 * NB:]
        wid = lax.axis_index("s") * NC + lax.axis_index("c")
        base = wid * per_w
        pltpu.sync_copy(x_hbm.at[pl.ds(base, per_w)], idx_v)

        def issue(g, buf, gsem):
            ivec = jnp.minimum(idx_v[pl.ds(g * GR, GR)], V - 1)
            for r in range(GR):
                pltpu.async_copy(ow_hbm.at[ivec[r]], buf.at[r], gsem)

        for k in range(NB):
            issue(k, bufs[k], gsems[k])

        for g in range(n_g):
            b = g % NB
            pltpu.make_async_copy(
                ow_hbm.at[pl.ds(0, GR)], bufs[b], gsems[b]).wait()
            pltpu.async_copy(
                bufs[b], out_hbm.at[pl.ds(base + g * GR, GR)], ssems[b])
            if g + NB < n_g:
                pltpu.make_async_copy(
                    bufs[b], out_hbm.at[pl.ds(0, GR)], ssems[b]).wait()
                issue(g + NB, bufs[b], gsems[b])

        for g in range(max(0, n_g - NB), n_g):
            b = g % NB
            pltpu.make_async_copy(
                bufs[b], out_hbm.at[pl.ds(0, GR)], ssems[b]).wait()

        def fix_body(h, _):
            ivec = idx_v[pl.ds(h * L, L)]
            for r in range(L):
                iv = ivec[r]

                @pl.when(iv >= V)
                def _():
                    pltpu.sync_copy(nw_hbm.at[iv - (V - 1)], bufs[0].at[0])
                    pltpu.sync_copy(
                        bufs[0].at[0], out_hbm.at[base + h * L + r])
            return 0

        lax.fori_loop(0, per_w // L, fix_body, 0)

    return gather


def _cvt_body(x_ref, o_ref):
    o_ref[...] = x_ref[...].astype(jnp.int32)


@functools.lru_cache(maxsize=None)
def _build_convert(B, D, blk):
    return pl.pallas_call(
        _cvt_body,
        grid=(pl.cdiv(B, blk),),
        in_specs=[pl.BlockSpec((blk, D), lambda i: (i, 0))],
        out_specs=pl.BlockSpec((blk, D), lambda i: (i, 0)),
        out_shape=jax.ShapeDtypeStruct((B, D), jnp.int32),
    )


def kernel(x, original_weight, new_weight):
    V, D = original_weight.shape
    Bt, S = x.shape
    B = Bt * S
    rows = _build_gather(V, D, B)(x.reshape(B), original_weight, new_weight)
    out = _build_convert(B, D, 512)(rows)
    return out.reshape(Bt, S, D)


# speedup vs baseline: 12.3134x; 1.1873x over previous
import functools

import jax
import jax.numpy as jnp
from jax import lax
from jax.experimental import pallas as pl
from jax.experimental.pallas import tpu as pltpu
from jax.experimental.pallas import tpu_sc as plsc


@functools.lru_cache(maxsize=None)
def _build_lookup(V, D, B, N1):
    info = plsc.get_sparse_core_info()
    NC, NS, L = info.num_cores, info.num_subcores, info.num_lanes
    NW = NC * NS
    assert B % NW == 0 and D % L == 0
    per_w = B // NW
    GR = L
    n_g = per_w // GR
    assert n_g % 2 == 0
    mesh = plsc.VectorSubcoreMesh(core_axis_name="c", subcore_axis_name="s")

    @functools.partial(
        pl.kernel,
        mesh=mesh,
        out_type=jax.ShapeDtypeStruct((B, D), jnp.int32),
        scratch_types=[
            pltpu.VMEM((per_w,), jnp.int32),
            pltpu.VMEM((GR, D), jnp.float32),
            pltpu.VMEM((GR, D), jnp.float32),
            pltpu.VMEM((GR, D), jnp.int32),
            pltpu.VMEM((GR, D), jnp.int32),
            pltpu.VMEM((1, D), jnp.float32),
            pltpu.VMEM((1, D), jnp.int32),
            pltpu.SemaphoreType.DMA,
            pltpu.SemaphoreType.DMA,
            pltpu.SemaphoreType.DMA,
            pltpu.SemaphoreType.DMA,
        ],
    )
    def lookup(x_hbm, ow_hbm, nw_hbm, out_hbm,
               idx_v, buf0, buf1, outb0, outb1, fbuf, fobuf,
               gsem0, gsem1, osem0, osem1):
        wid = lax.axis_index("s") * NC + lax.axis_index("c")
        base = wid * per_w
        pltpu.sync_copy(x_hbm.at[pl.ds(base, per_w)], idx_v)

        def issue(g, buf, gsem):
            ivec = jnp.minimum(idx_v[pl.ds(g * GR, GR)], V - 1)
            for r in range(GR):
                pltpu.async_copy(ow_hbm.at[ivec[r]], buf.at[r], gsem)

        def wait_rows(buf, gsem):
            pltpu.make_async_copy(ow_hbm.at[pl.ds(0, GR)], buf, gsem).wait()

        def convert(buf, outb):
            for r in range(GR):
                for c in range(D // L):
                    cs = pl.ds(c * L, L)
                    outb[r, cs] = buf[r, cs].astype(jnp.int32)

        def half(i, g, buf, outb, gsem, osem):
            wait_rows(buf, gsem)

            @pl.when(i >= 1)
            def _():
                pltpu.make_async_copy(
                    outb, out_hbm.at[pl.ds(0, GR)], osem).wait()

            convert(buf, outb)
            pltpu.async_copy(outb, out_hbm.at[pl.ds(base + g * GR, GR)], osem)

            @pl.when(g + 2 < n_g)
            def _():
                issue(g + 2, buf, gsem)

        issue(0, buf0, gsem0)
        issue(1, buf1, gsem1)

        def pair_body(i, _):
            half(i, 2 * i, buf0, outb0, gsem0, osem0)
            half(i, 2 * i + 1, buf1, outb1, gsem1, osem1)
            return 0

        lax.fori_loop(0, n_g // 2, pair_body, 0)
        pltpu.make_async_copy(outb0, out_hbm.at[pl.ds(0, GR)], osem0).wait()
        pltpu.make_async_copy(outb1, out_hbm.at[pl.ds(0, GR)], osem1).wait()

        def fix_body(h, _):
            ivec = idx_v[pl.ds(h * L, L)]
            for r in range(L):
                iv = ivec[r]

                @pl.when(iv >= V)
                def _():
                    pltpu.sync_copy(nw_hbm.at[iv - (V - 1)], fbuf.at[0])
                    for c in range(D // L):
                        cs = pl.ds(c * L, L)
                        fobuf[0, cs] = fbuf[0, cs].astype(jnp.int32)
                    pltpu.sync_copy(
                        fobuf.at[0], out_hbm.at[base + h * L + r])
            return 0

        lax.fori_loop(0, per_w // L, fix_body, 0)

    return lookup


def kernel(x, original_weight, new_weight):
    V, D = original_weight.shape
    N1 = new_weight.shape[0]
    Bt, S = x.shape
    B = Bt * S
    lookup = _build_lookup(V, D, B, N1)
    out = lookup(x.reshape(B), original_weight, new_weight)
    return out.reshape(Bt, S, D)
